# baseline (speedup 1.0000x reference)
import functools

import numpy as np
import jax
import jax.numpy as jnp
from jax import lax
from jax.experimental import pallas as pl
from jax.experimental.pallas import tpu as pltpu
from jax.experimental.pallas import tpu_sc as plsc

N_LEVELS = 16
TABLE_SIZE = 524288
FEATS = 2
BASE_RES = 16
MAX_RES = 2048
_growth = (MAX_RES / BASE_RES) ** (1.0 / (N_LEVELS - 1))
RES_LIST = [int(BASE_RES * _growth**l) for l in range(N_LEVELS)]
PRIME_Y = np.uint32(2654435761).astype(np.int32).item()
PRIME_Z = np.uint32(805459861).astype(np.int32).item()
HASH_MASK = TABLE_SIZE - 1
LEVEL_FLOATS = TABLE_SIZE * FEATS

NUM_WORKERS = 32


def _build_sc_kernel(n_pos: int, chunk: int):
    per_w = n_pos // NUM_WORKERS
    n_chunks = per_w // chunk
    assert per_w % chunk == 0 and chunk % 128 == 0
    j_rows = chunk // 128
    n_groups = chunk // 16
    asm_groups = chunk * FEATS // 16

    mesh = plsc.VectorSubcoreMesh(
        core_axis_name="c", subcore_axis_name="s", num_cores=2, num_subcores=16
    )

    @functools.partial(
        pl.kernel,
        mesh=mesh,
        out_type=jax.ShapeDtypeStruct((4, n_pos // 128, 8, 128), jnp.float32),
        scratch_types=[
            pltpu.VMEM((chunk,), jnp.float32),
            pltpu.VMEM((chunk,), jnp.float32),
            pltpu.VMEM((chunk,), jnp.float32),
            pltpu.VMEM((N_LEVELS * j_rows, 128), jnp.int32),
            pltpu.VMEM((N_LEVELS * j_rows, 128), jnp.int32),
            pltpu.VMEM((N_LEVELS * j_rows, 128), jnp.int32),
            pltpu.VMEM((2, 2, j_rows, 128, 8), jnp.float32),
            pltpu.VMEM((4, chunk // 128, 8, 128), jnp.float32),
            pltpu.SemaphoreType.DMA,
            pltpu.SemaphoreType.DMA,
        ],
        compiler_params=pltpu.CompilerParams(
            needs_layout_passes=False, use_tc_tiling_on_sc=False
        ),
    )
    def sc_kernel(px_hbm, py_hbm, pz_hbm, tab_hbm, out_hbm,
                  px_v, py_v, pz_v, idx_v, sup0_v, sup1_v, gath_v, out_v,
                  sem0, sem1):
        wid = lax.axis_index("s") * 2 + lax.axis_index("c")
        lanes = lax.iota(jnp.int32, 16)
        half = lanes >> 1
        parity = lanes & 1

        def fire(l, b, sem):
            def body(j, c):
                row = l * j_rows + j
                pltpu.async_copy(
                    tab_hbm.at[sup0_v.at[row]], gath_v.at[b, 0, j], sem
                )
                pltpu.async_copy(
                    tab_hbm.at[sup1_v.at[row]], gath_v.at[b, 1, j], sem
                )
                return c
            lax.fori_loop(0, j_rows, body, 0, unroll=False)

        def drain(l, b, sem):
            def body(j, c):
                row = l * j_rows + j
                pltpu.make_async_copy(
                    tab_hbm.at[sup0_v.at[row]], gath_v.at[b, 0, j], sem
                ).wait()
                pltpu.make_async_copy(
                    tab_hbm.at[sup1_v.at[row]], gath_v.at[b, 1, j], sem
                ).wait()
                return c
            lax.fori_loop(0, j_rows, body, 0, unroll=False)

        def chunk_body(ci, carry):
            pbase = wid * per_w + ci * chunk
            pltpu.sync_copy(px_hbm.at[pl.ds(pbase, chunk)], px_v)
            pltpu.sync_copy(py_hbm.at[pl.ds(pbase, chunk)], py_v)
            pltpu.sync_copy(pz_hbm.at[pl.ds(pbase, chunk)], pz_v)

            def hash_body(g, c2):
                sl = pl.ds(g * 16, 16)
                px = jnp.maximum(px_v[sl], 0.0)
                py = jnp.maximum(py_v[sl], 0.0)
                pz = jnp.maximum(pz_v[sl], 0.0)
                jrow = g >> 3
                col = (g & 7) * 16
                for l in range(N_LEVELS):
                    res = RES_LIST[l]
                    cx = jnp.minimum((px * res).astype(jnp.int32), res - 1)
                    cy = jnp.minimum((py * res).astype(jnp.int32), res - 1)
                    cz = jnp.minimum((pz * res).astype(jnp.int32), res - 1)
                    h = (cx + cy * PRIME_Y + cz * PRIME_Z) & HASH_MASK
                    flat = ((h >> 7) << 8) | (h & 127)
                    idx_v[l * j_rows + jrow, pl.ds(col, 16)] = flat + l * LEVEL_FLOATS
                return c2

            lax.fori_loop(0, n_groups, hash_body, 0, unroll=2)

            def sup_body(t, c2):
                row = t >> 3
                col = pl.ds((t & 7) * 16, 16)
                s0 = idx_v[row, col] >> 3
                sup0_v[row, col] = s0
                sup1_v[row, col] = s0 + 16
                return c2

            lax.fori_loop(0, N_LEVELS * j_rows * 8, sup_body, 0, unroll=4)

            def asm_level(l, b):
                def asm(q, c3):
                    jrow = q >> 4
                    r0 = (q & 15) * 8
                    row = jnp.full((16,), l * j_rows + jrow, jnp.int32)
                    jsplat = jnp.full((16,), jrow, jnp.int32)
                    f16 = plsc.load_gather(idx_v, [row, r0 + half])
                    sub = f16 & 7
                    x = plsc.load_gather(
                        gath_v, [jnp.full((16,), b, jnp.int32), parity,
                                 jsplat, r0 + half, sub]
                    )
                    c = 2 * l + parity
                    plsc.store_scatter(
                        out_v, [c >> 3, jsplat, c & 7, r0 + half], x
                    )
                    return c3

                lax.fori_loop(0, asm_groups, asm, 0, unroll=False)

            sems = (sem0, sem1)
            fire(0, 0, sem0)
            for l in range(N_LEVELS):
                if l + 1 < N_LEVELS:
                    fire(l + 1, (l + 1) & 1, sems[(l + 1) & 1])
                drain(l, l & 1, sems[l & 1])
                asm_level(l, l & 1)
            pblk = pbase // 128
            for fb in range(4):
                pltpu.sync_copy(
                    out_v.at[fb], out_hbm.at[fb, pl.ds(pblk, chunk // 128)]
                )
            return carry

        lax.fori_loop(0, n_chunks, chunk_body, 0, unroll=False)

    return sc_kernel


def kernel(positions, tables):
    n_pos = positions.shape[0]
    px = positions[:, 0]
    py = positions[:, 1]
    pz = positions[:, 2]
    tab_flat = (tables.reshape(N_LEVELS, 4096, 128, 2)
                .swapaxes(2, 3).reshape(N_LEVELS * TABLE_SIZE // 4, 8))
    fn = _build_sc_kernel(n_pos, chunk=1024)
    out4 = fn(px, py, pz, tab_flat)
    return out4.transpose(1, 3, 0, 2).reshape(n_pos, N_LEVELS * FEATS)


# speedup vs baseline: 1.0210x; 1.0210x over previous
import functools

import numpy as np
import jax
import jax.numpy as jnp
from jax import lax
from jax.experimental import pallas as pl
from jax.experimental.pallas import tpu as pltpu
from jax.experimental.pallas import tpu_sc as plsc

N_LEVELS = 16
TABLE_SIZE = 524288
FEATS = 2
BASE_RES = 16
MAX_RES = 2048
_growth = (MAX_RES / BASE_RES) ** (1.0 / (N_LEVELS - 1))
RES_LIST = [int(BASE_RES * _growth**l) for l in range(N_LEVELS)]
PRIME_Y = np.uint32(2654435761).astype(np.int32).item()
PRIME_Z = np.uint32(805459861).astype(np.int32).item()
HASH_MASK = TABLE_SIZE - 1
LEVEL_FLOATS = TABLE_SIZE * FEATS

NUM_WORKERS = 32


def _build_sc_kernel(n_pos: int, chunk: int):
    per_w = n_pos // NUM_WORKERS
    n_chunks = per_w // chunk
    assert per_w % chunk == 0 and chunk % 128 == 0
    j_rows = chunk // 128
    n_groups = chunk // 16
    asm_groups = chunk * FEATS // 16

    mesh = plsc.VectorSubcoreMesh(
        core_axis_name="c", subcore_axis_name="s", num_cores=2, num_subcores=16
    )

    @functools.partial(
        pl.kernel,
        mesh=mesh,
        out_type=jax.ShapeDtypeStruct((4, n_pos // 128, 8, 128), jnp.float32),
        scratch_types=[
            pltpu.VMEM((chunk,), jnp.float32),
            pltpu.VMEM((chunk,), jnp.float32),
            pltpu.VMEM((chunk,), jnp.float32),
            pltpu.VMEM((N_LEVELS * j_rows, 128), jnp.int32),
            pltpu.VMEM((N_LEVELS * j_rows, 128), jnp.int32),
            pltpu.VMEM((N_LEVELS * j_rows, 128), jnp.int32),
            pltpu.VMEM((2, 2, j_rows, 128, 8), jnp.float32),
            pltpu.VMEM((4, chunk // 128, 8, 128), jnp.float32),
            pltpu.SemaphoreType.DMA,
            pltpu.SemaphoreType.DMA,
        ],
        compiler_params=pltpu.CompilerParams(
            needs_layout_passes=False, use_tc_tiling_on_sc=False
        ),
    )
    def sc_kernel(px_hbm, py_hbm, pz_hbm, tab_hbm, out_hbm,
                  px_v, py_v, pz_v, idx_v, sup0_v, sup1_v, gath_v, out_v,
                  sem0, sem1):
        wid = lax.axis_index("s") * 2 + lax.axis_index("c")
        lanes = lax.iota(jnp.int32, 16)
        half = lanes >> 1
        parity = lanes & 1

        def fire(l, b, sem):
            def body(j, c):
                row = l * j_rows + j
                pltpu.async_copy(
                    tab_hbm.at[sup0_v.at[row]], gath_v.at[b, 0, j], sem
                )
                pltpu.async_copy(
                    tab_hbm.at[sup1_v.at[row]], gath_v.at[b, 1, j], sem
                )
                return c
            lax.fori_loop(0, j_rows, body, 0, unroll=False)

        def drain(l, b, sem):
            def body(j, c):
                row = l * j_rows + j
                pltpu.make_async_copy(
                    tab_hbm.at[sup0_v.at[row]], gath_v.at[b, 0, j], sem
                ).wait()
                pltpu.make_async_copy(
                    tab_hbm.at[sup1_v.at[row]], gath_v.at[b, 1, j], sem
                ).wait()
                return c
            lax.fori_loop(0, j_rows, body, 0, unroll=False)

        def chunk_body(ci, carry):
            pbase = wid * per_w + ci * chunk
            pltpu.sync_copy(px_hbm.at[pl.ds(pbase, chunk)], px_v)
            pltpu.sync_copy(py_hbm.at[pl.ds(pbase, chunk)], py_v)
            pltpu.sync_copy(pz_hbm.at[pl.ds(pbase, chunk)], pz_v)

            def hash_body(g, c2):
                sl = pl.ds(g * 16, 16)
                px = jnp.maximum(px_v[sl], 0.0)
                py = jnp.maximum(py_v[sl], 0.0)
                pz = jnp.maximum(pz_v[sl], 0.0)
                jrow = g >> 3
                col = (g & 7) * 16
                for l in range(N_LEVELS):
                    res = RES_LIST[l]
                    cx = jnp.minimum((px * res).astype(jnp.int32), res - 1)
                    cy = jnp.minimum((py * res).astype(jnp.int32), res - 1)
                    cz = jnp.minimum((pz * res).astype(jnp.int32), res - 1)
                    h = (cx + cy * PRIME_Y + cz * PRIME_Z) & HASH_MASK
                    flat = ((h >> 7) << 8) | (h & 127)
                    idx_v[l * j_rows + jrow, pl.ds(col, 16)] = flat + l * LEVEL_FLOATS
                return c2

            lax.fori_loop(0, n_groups, hash_body, 0, unroll=False)

            def sup_body(t, c2):
                row = t >> 3
                col = pl.ds((t & 7) * 16, 16)
                s0 = idx_v[row, col] >> 3
                sup0_v[row, col] = s0
                sup1_v[row, col] = s0 + 16
                return c2

            lax.fori_loop(0, N_LEVELS * j_rows * 8, sup_body, 0, unroll=False)

            def asm_level(l, b):
                def asm(q, c3):
                    jrow = q >> 4
                    r0 = (q & 15) * 8
                    row = jnp.full((16,), l * j_rows + jrow, jnp.int32)
                    jsplat = jnp.full((16,), jrow, jnp.int32)
                    f16 = plsc.load_gather(idx_v, [row, r0 + half])
                    sub = f16 & 7
                    x = plsc.load_gather(
                        gath_v, [jnp.full((16,), b, jnp.int32), parity,
                                 jsplat, r0 + half, sub]
                    )
                    c = 2 * l + parity
                    plsc.store_scatter(
                        out_v, [c >> 3, jsplat, c & 7, r0 + half], x
                    )
                    return c3

                lax.fori_loop(0, asm_groups, asm, 0, unroll=False)

            sems = (sem0, sem1)
            fire(0, 0, sem0)
            for l in range(N_LEVELS):
                if l + 1 < N_LEVELS:
                    fire(l + 1, (l + 1) & 1, sems[(l + 1) & 1])
                drain(l, l & 1, sems[l & 1])
                asm_level(l, l & 1)
            pblk = pbase // 128
            for fb in range(4):
                pltpu.sync_copy(
                    out_v.at[fb], out_hbm.at[fb, pl.ds(pblk, chunk // 128)]
                )
            return carry

        lax.fori_loop(0, n_chunks, chunk_body, 0, unroll=False)

    return sc_kernel


def kernel(positions, tables):
    n_pos = positions.shape[0]
    px = positions[:, 0]
    py = positions[:, 1]
    pz = positions[:, 2]
    tab_flat = (tables.reshape(N_LEVELS, 4096, 128, 2)
                .swapaxes(2, 3).reshape(N_LEVELS * TABLE_SIZE // 4, 8))
    fn = _build_sc_kernel(n_pos, chunk=1024)
    out4 = fn(px, py, pz, tab_flat)
    return out4.transpose(1, 3, 0, 2).reshape(n_pos, N_LEVELS * FEATS)
